# baseline (speedup 1.0000x reference)
import jax
import jax.numpy as jnp
from jax import lax
from jax.experimental import pallas as pl
from jax.experimental.pallas import tpu as pltpu
from jax.experimental.pallas import tpu_sc as plsc

N = 10000
E = 320000
D = 128

NC = 2
NS = 16
NW = NC * NS
CH = 128
NCH = 80
EPW = NCH * CH
E_PAD = EPW * NW
ACC_ROWS = N + 8
ST = 640
PH = NCH // 2

_MB = 1000


def _agg_body(src_hbm, dst_hbm, m_hbm, out_hbm,
              sstage, dstage, r0, r1, acc, semi, sg0, sg1, ss0, ss1):
    rows = [r0, r1]
    semg = [sg0, sg1]
    sems = [ss0, ss1]
    c = lax.axis_index("c")
    s = lax.axis_index("s")
    w = c * NS + s

    zero16 = jnp.zeros((16,), jnp.float32)

    def _zfill(i, carry):
        for j in range(8):
            r0[i, pl.ds(j * 16, 16)] = zero16
        return carry

    lax.fori_loop(0, CH, _zfill, 0)
    ofs = pl.multiple_of(s * ST, 8)

    @pl.when(s < 15)
    def _():
        for k in range(ST // CH):
            pltpu.sync_copy(r0, acc.at[pl.ds(ofs + k * CH, CH), :])

    @pl.when(s == 15)
    def _():
        for k in range(3):
            pltpu.sync_copy(r0, acc.at[pl.ds(15 * ST + k * CH, CH), :])
        pltpu.sync_copy(r0.at[pl.ds(0, 24), :],
                        acc.at[pl.ds(15 * ST + 3 * CH, 24), :])

    pltpu.async_copy(src_hbm.at[w], sstage, semi)
    pltpu.make_async_copy(src_hbm.at[0], sstage, semi).wait()
    plsc.subcore_barrier()

    def gather(g, b):
        pltpu.async_copy(m_hbm.at[sstage.at[g]], rows[b], semg[b])

    def wait_gather(b):
        pltpu.make_async_copy(m_hbm.at[pl.ds(0, CH), :], rows[b],
                              semg[b]).wait()

    def scatter(gl, b):
        pltpu.async_copy(rows[b], acc.at[dstage.at[gl]], sems[b], add=True)

    def wait_scatter(b):
        pltpu.make_async_copy(rows[b], acc.at[pl.ds(0, CH), :],
                              sems[b]).wait()

    for p in range(2):
        base = pl.multiple_of(p * PH, 8)
        pltpu.async_copy(dst_hbm.at[w, pl.ds(base, PH), :], dstage, semi)
        pltpu.make_async_copy(dst_hbm.at[0, pl.ds(0, PH), :], dstage,
                              semi).wait()
        g0 = p * PH
        gather(g0 + 0, 0)
        gather(g0 + 1, 1)
        wait_gather(0)
        scatter(0, 0)

        def _pair(j, carry):
            wait_scatter(0)
            gather(g0 + 2 * j + 2, 0)
            wait_gather(1)
            scatter(2 * j + 1, 1)
            wait_scatter(1)
            gather(g0 + 2 * j + 3, 1)
            wait_gather(0)
            scatter(2 * j + 2, 0)
            return carry

        lax.fori_loop(0, PH // 2 - 1, _pair, 0)
        wait_gather(1)
        scatter(PH - 1, 1)
        wait_scatter(0)
        wait_scatter(1)

    plsc.subcore_barrier()

    @pl.when(s < 15)
    def _():
        pltpu.sync_copy(acc.at[pl.ds(ofs, ST), :],
                        out_hbm.at[c, pl.ds(ofs, ST), :])

    @pl.when(s == 15)
    def _():
        pltpu.sync_copy(acc.at[pl.ds(15 * ST, N - 15 * ST), :],
                        out_hbm.at[c, pl.ds(15 * ST, N - 15 * ST), :])


@jax.jit
def _agg(src3, dst3, m):
    mesh = plsc.VectorSubcoreMesh(core_axis_name="c", subcore_axis_name="s")
    row_t = pltpu.VMEM((CH, D), jnp.float32)
    return pl.kernel(
        _agg_body,
        out_type=jax.ShapeDtypeStruct((NC, N, D), jnp.float32),
        mesh=mesh,
        scratch_types=[
            pltpu.VMEM((NCH, CH), jnp.int32),
            pltpu.VMEM((PH, CH), jnp.int32),
            row_t, row_t,
            pltpu.VMEM_SHARED((ACC_ROWS, D), jnp.float32),
        ] + [pltpu.SemaphoreType.DMA] * 5,
    )(src3, dst3, m)


def _mm_body(x_ref, w_ref, o_ref):
    o_ref[...] = jnp.dot(x_ref[...], w_ref[...],
                         preferred_element_type=jnp.float32)


@jax.jit
def _mm(x, w):
    return pl.pallas_call(
        _mm_body,
        grid=(N // _MB,),
        in_specs=[
            pl.BlockSpec((_MB, D), lambda i: (i, 0)),
            pl.BlockSpec((D, D), lambda i: (0, 0)),
        ],
        out_specs=pl.BlockSpec((_MB, D), lambda i: (i, 0)),
        out_shape=jax.ShapeDtypeStruct((N, D), jnp.float32),
    )(x, w)


def _combine_mm_body(p_ref, b_ref, w_ref, o_ref):
    x = jnp.maximum(p_ref[0] + p_ref[1] + b_ref[...], 0.0)
    o_ref[...] = jnp.dot(x, w_ref[...], preferred_element_type=jnp.float32)


@jax.jit
def _combine_mm(p, b, w):
    return pl.pallas_call(
        _combine_mm_body,
        grid=(N // _MB,),
        in_specs=[
            pl.BlockSpec((NC, _MB, D), lambda i: (0, i, 0)),
            pl.BlockSpec((1, D), lambda i: (0, 0)),
            pl.BlockSpec((D, D), lambda i: (0, 0)),
        ],
        out_specs=pl.BlockSpec((_MB, D), lambda i: (i, 0)),
        out_shape=jax.ShapeDtypeStruct((N, D), jnp.float32),
    )(p, b, w)


def _head_body(p_ref, b2_ref, wh1_ref, bh1_ref, wh2_ref, bh2_ref,
               wc_ref, bc_ref, o_ref):
    x = jnp.maximum(p_ref[0] + p_ref[1] + b2_ref[...], 0.0)
    h = jnp.maximum(
        jnp.dot(x, wh1_ref[...], preferred_element_type=jnp.float32)
        + bh1_ref[...], 0.0)
    h = jnp.maximum(
        jnp.dot(h, wh2_ref[...], preferred_element_type=jnp.float32)
        + bh2_ref[...], 0.0)
    av = (jnp.dot(h, wc_ref[...], preferred_element_type=jnp.float32)
          + bc_ref[...])
    col = lax.broadcasted_iota(jnp.int32, av.shape, 1)
    adv_sum = jnp.sum(jnp.where(col < 5, av, 0.0), axis=1, keepdims=True)
    val = jnp.sum(jnp.where(col == 5, av, 0.0), axis=1, keepdims=True)
    o_ref[...] = val + av - adv_sum * (1.0 / 5.0)


@jax.jit
def _head(p, b2, wh1, bh1, wh2, bh2, wc, bc):
    return pl.pallas_call(
        _head_body,
        grid=(N // _MB,),
        in_specs=[
            pl.BlockSpec((NC, _MB, D), lambda i: (0, i, 0)),
            pl.BlockSpec((1, D), lambda i: (0, 0)),
            pl.BlockSpec((D, D), lambda i: (0, 0)),
            pl.BlockSpec((1, D), lambda i: (0, 0)),
            pl.BlockSpec((D, D), lambda i: (0, 0)),
            pl.BlockSpec((1, D), lambda i: (0, 0)),
            pl.BlockSpec((D, 8), lambda i: (0, 0)),
            pl.BlockSpec((1, 8), lambda i: (0, 0)),
        ],
        out_specs=pl.BlockSpec((_MB, 8), lambda i: (i, 0)),
        out_shape=jax.ShapeDtypeStruct((N, 8), jnp.float32),
    )(p, b2, wh1, bh1, wh2, bh2, wc, bc)


def kernel(edge_index, entity_embeddings, W1, b1, W2, b2,
           Wh1, bh1, Wh2, bh2, Wadv, badv, Wval, bval):
    pad = E_PAD - E
    src = jnp.concatenate([edge_index[0], jnp.zeros((pad,), jnp.int32)])
    dst = jnp.concatenate([edge_index[1], jnp.full((pad,), N, jnp.int32)])
    src = src.reshape(NW, NCH, CH)
    dst = dst.reshape(NW, NCH, CH)
    wc = jnp.concatenate([Wadv, Wval, jnp.zeros((D, 2), jnp.float32)], axis=1)
    bc = jnp.concatenate([badv, bval, jnp.zeros((2,), jnp.float32)])[None, :]

    m1 = _mm(entity_embeddings, W1)
    p1 = _agg(src, dst, m1)
    m2 = _combine_mm(p1, b1[None, :], W2)
    p2 = _agg(src, dst, m2)
    q8 = _head(p2, b2[None, :], Wh1, bh1[None, :], Wh2, bh2[None, :], wc, bc)
    return q8[:, :5]
